# baseline (speedup 1.0000x reference)
import functools

import jax
import jax.numpy as jnp
from jax.experimental import pallas as pl
from jax.experimental.pallas import tpu as pltpu
from jax.experimental.pallas import tpu_sc as plsc

TOKENS = 2048
INPUT = 1024
EMB = 512
NB = 8
NJ = 4
NCL = 128
B = 128
NT = 24
SORT = NT * B
PADN = SORT + B
OUTPAD = TOKENS + B
TOKW = 128
NW = 16
L = 16

_relu = lambda z: jnp.maximum(z, 0.0)


def _dot(a, b):
    return jnp.dot(a, b, preferred_element_type=jnp.float32)


def _addr_flat(logits):
    a0 = (logits[:, 1:2] > logits[:, 0:1]).astype(jnp.int32)
    a1 = (logits[:, 3:4] > logits[:, 2:3]).astype(jnp.int32)
    a2 = (logits[:, 5:6] > logits[:, 4:5]).astype(jnp.int32)
    return 4 * a0 + 2 * a1 + a2


def _embed_body(x_ref, We_ref, be_ref, Wa1_ref, ba1_ref, Wa2_ref, ba2_ref,
                state_ref, idx_ref):
    state = _dot(x_ref[...], We_ref[...]) + be_ref[...]
    state_ref[...] = state
    h0 = _relu(_dot(state, Wa1_ref[0]) + ba1_ref[0])
    logits = _dot(h0, Wa2_ref[0]) + ba2_ref[0]
    idx_ref[...] = _addr_flat(logits).reshape(1, B, 1)


def _embed(x, W_emb, b_emb, Wa1, ba1, Wa2p, ba2p):
    full = lambda r: pl.BlockSpec(None, lambda i: (0,) * r)
    return pl.pallas_call(
        _embed_body,
        grid=(TOKENS // B,),
        in_specs=[pl.BlockSpec((B, INPUT), lambda i: (i, 0)),
                  full(2), full(1), full(3), full(2), full(3), full(2)],
        out_specs=[pl.BlockSpec((B, EMB), lambda i: (i, 0)),
                   pl.BlockSpec((1, B, 1), lambda i: (i, 0, 0))],
        out_shape=[jax.ShapeDtypeStruct((TOKENS, EMB), jnp.float32),
                   jax.ShapeDtypeStruct((TOKENS // B, B, 1), jnp.int32)],
    )(x, W_emb, b_emb, Wa1, ba1, Wa2p, ba2p)


def _lane_scalar(vec, e):
    iot = jax.lax.iota(jnp.int32, L)
    return jnp.sum(jnp.where(iot == e, vec, 0))


def _make_hist(n_in):
    nslots = n_in // NW
    nch = nslots // L
    scratch = [pltpu.VMEM((nslots,), jnp.int32),
               pltpu.VMEM((L,), jnp.int32)]

    def body(idx_hbm, ohist, idx_v, histbuf):
        wid = jax.lax.axis_index("s")
        base = wid * nslots
        iot = jax.lax.iota(jnp.int32, L)
        pltpu.sync_copy(idx_hbm.at[pl.ds(base, nslots)], idx_v)
        hist = jnp.zeros((L,), jnp.int32)
        for c in range(nch):
            v = idx_v[pl.ds(c * L, L)]
            for e in range(NB):
                cnt = jnp.sum((v == e).astype(jnp.int32))
                hist = hist + jnp.where(iot == e, cnt, 0)
        histbuf[...] = hist
        pltpu.sync_copy(histbuf, ohist.at[wid])

    mesh = plsc.VectorSubcoreMesh(core_axis_name="c", subcore_axis_name="s",
                                  num_cores=1)
    return pl.kernel(body,
                     out_type=jax.ShapeDtypeStruct((NW, L), jnp.int32),
                     mesh=mesh, scratch_types=tuple(scratch),
                     compiler_params=pltpu.CompilerParams(
                         needs_layout_passes=False))


def _make_route(n_in, gen_tokid, with_init):
    nslots = n_in // NW
    nch = nslots // L
    outs = [jax.ShapeDtypeStruct((PADN, EMB), jnp.float32),
            jax.ShapeDtypeStruct((PADN, TOKW), jnp.int32),
            jax.ShapeDtypeStruct((96,), jnp.int32)]
    if with_init:
        outs.append(jax.ShapeDtypeStruct((PADN, EMB), jnp.float32))
    scratch = [pltpu.VMEM((nslots,), jnp.int32),
               pltpu.VMEM((nslots,), jnp.int32),
               pltpu.VMEM((NW, L), jnp.int32),
               pltpu.VMEM((L, EMB), jnp.float32),
               pltpu.VMEM((L, TOKW), jnp.int32),
               pltpu.VMEM((96,), jnp.int32),
               pltpu.VMEM((L,), jnp.int32),
               pltpu.VMEM((L,), jnp.int32),
               pltpu.SemaphoreType.DMA]
    if with_init:
        scratch.append(pltpu.VMEM((L, EMB), jnp.float32))

    def body(*refs):
        i = 0
        idx_hbm = refs[0]; rows_hbm = refs[1]; i = 2
        tok_hbm = None
        init_hbm = None
        if not gen_tokid:
            tok_hbm = refs[i]; i += 1
        if with_init:
            init_hbm = refs[i]; i += 1
        hists_hbm = refs[i]; i += 1
        operm = refs[i]; otok = refs[i + 1]; ometa = refs[i + 2]; i += 3
        if with_init:
            oinit = refs[i]; i += 1
        (idx_v, pos_v, allhist, rowbuf, tokbuf, metabuf,
         posbuf, tidbuf, sem) = refs[i:i + 9]
        initbuf = refs[i + 9] if with_init else None

        wid = jax.lax.axis_index("s")
        base = wid * nslots
        iot = jax.lax.iota(jnp.int32, L)
        zer = jnp.zeros((L,), jnp.int32)

        pltpu.sync_copy(idx_hbm.at[pl.ds(base, nslots)], idx_v)
        pltpu.sync_copy(hists_hbm, allhist)

        g = jnp.zeros((L,), jnp.int32)
        pre = jnp.zeros((L,), jnp.int32)
        for w in range(NW):
            row = allhist[w, :]
            g = g + row
            pre = pre + jnp.where(jnp.int32(w) < wid, row, 0)

        lane8 = (iot < NB).astype(jnp.int32)
        counts = g * lane8
        T = ((counts + B - 1) >> 7) * lane8
        ts = plsc.cumsum(T) - T
        n_used = jnp.sum(T)
        mybase = ts * B + pre
        base_sc = [_lane_scalar(mybase, e) for e in range(NB)]
        dump = jnp.int32(SORT) + wid

        run = [jnp.int32(0)] * NB
        for c in range(nch):
            v = idx_v[pl.ds(c * L, L)]
            pos = jnp.full((L,), 0, jnp.int32) + dump
            for e in range(NB):
                m = v == e
                mi = m.astype(jnp.int32)
                r = plsc.cumsum(mi) - 1
                pos = jnp.where(m, base_sc[e] + run[e] + r, pos)
                run[e] = run[e] + jnp.sum(mi)
            pos_v[pl.ds(c * L, L)] = pos

        @pl.when(wid == 0)
        def _():
            ts_sc = [_lane_scalar(ts, e) for e in range(NB)]
            cnt_sc = [_lane_scalar(counts, e) for e in range(NB)]
            k0 = iot
            k1 = iot + L
            e0 = jnp.full((L,), -1, jnp.int32)
            e1 = jnp.full((L,), -1, jnp.int32)
            for e in range(NB):
                e0 = e0 + (ts_sc[e] <= k0).astype(jnp.int32)
                e1 = e1 + (ts_sc[e] <= k1).astype(jnp.int32)
            e0 = jnp.clip(e0, 0, NB - 1)
            e1 = jnp.clip(e1, 0, NB - 1)
            v0 = zer
            v1 = zer
            for e in range(NB):
                c0 = jnp.clip(cnt_sc[e] - (k0 - ts_sc[e]) * B, 0, B)
                c1 = jnp.clip(cnt_sc[e] - (k1 - ts_sc[e]) * B, 0, B)
                v0 = jnp.where(e0 == e, c0, v0)
                v1 = jnp.where(e1 == e, c1, v1)
            metabuf[pl.ds(0, L)] = e0
            metabuf[pl.ds(16, L)] = e1
            metabuf[pl.ds(32, L)] = v0
            metabuf[pl.ds(48, L)] = v1
            metabuf[pl.ds(64, L)] = zer
            metabuf[pl.ds(80, L)] = zer + n_used
            pltpu.sync_copy(metabuf, ometa)

        for c in range(nch):
            pv = jnp.clip(pos_v[pl.ds(c * L, L)], 0, PADN - 1)
            posbuf[...] = pv
            pltpu.sync_copy(rows_hbm.at[pl.ds(base + c * L, L)], rowbuf)
            pltpu.async_copy(rowbuf, operm.at[posbuf], sem).wait()
            if gen_tokid:
                tv = base + c * L + iot
                plsc.store_scatter(tokbuf, [iot, zer], tv)
            else:
                pltpu.sync_copy(tok_hbm.at[pl.ds(base + c * L, L)], tokbuf)
                tv = plsc.load_gather(tokbuf, [iot, zer])
            pltpu.async_copy(tokbuf, otok.at[posbuf], sem).wait()
            if with_init:
                vi = idx_v[pl.ds(c * L, L)]
                tidbuf[...] = jnp.clip(jnp.where(vi < NB, tv, 0), 0, TOKENS - 1)
                pltpu.async_copy(init_hbm.at[tidbuf], initbuf, sem).wait()
                pltpu.async_copy(initbuf, oinit.at[posbuf], sem).wait()

    mesh = plsc.VectorSubcoreMesh(core_axis_name="c", subcore_axis_name="s",
                                  num_cores=1)
    return pl.kernel(body, out_type=tuple(outs), mesh=mesh,
                     scratch_types=tuple(scratch),
                     compiler_params=pltpu.CompilerParams(
                         needs_layout_passes=False))


def _make_unsort():
    nslots = SORT // NW
    nch = nslots // L
    scratch = [pltpu.VMEM((nslots,), jnp.int32),
               pltpu.VMEM((L, EMB), jnp.float32),
               pltpu.VMEM((L, TOKW), jnp.int32),
               pltpu.VMEM((L,), jnp.int32),
               pltpu.SemaphoreType.DMA]

    def body(idx_hbm, rows_hbm, tok_hbm, out_hbm, idx_v, rowbuf, tokbuf,
             tidbuf, sem):
        wid = jax.lax.axis_index("s")
        base = wid * nslots
        iot = jax.lax.iota(jnp.int32, L)
        zer = jnp.zeros((L,), jnp.int32)
        pltpu.sync_copy(idx_hbm.at[pl.ds(base, nslots)], idx_v)
        for c in range(nch):
            v = idx_v[pl.ds(c * L, L)]
            pltpu.sync_copy(tok_hbm.at[pl.ds(base + c * L, L)], tokbuf)
            tv = plsc.load_gather(tokbuf, [iot, zer])
            tidbuf[...] = jnp.clip(jnp.where(v < NB, tv, jnp.int32(TOKENS) + wid),
                                   0, OUTPAD - 1)
            pltpu.sync_copy(rows_hbm.at[pl.ds(base + c * L, L)], rowbuf)
            pltpu.async_copy(rowbuf, out_hbm.at[tidbuf], sem).wait()

    mesh = plsc.VectorSubcoreMesh(core_axis_name="c", subcore_axis_name="s",
                                  num_cores=1)
    return pl.kernel(body,
                     out_type=jax.ShapeDtypeStruct((OUTPAD, EMB), jnp.float32),
                     mesh=mesh, scratch_types=tuple(scratch),
                     compiler_params=pltpu.CompilerParams(
                         needs_layout_passes=False))


def _jump_call(meta, perm, init_s, Ws1, bs1, Ws2, bs2, Wa1, ba1, Wa2p, ba2p,
               rw, last):
    def body(*refs):
        meta_ref = refs[0]
        i = 1
        perm_ref = refs[i]; i += 1
        init_ref = None
        if rw != 0.0:
            init_ref = refs[i]; i += 1
        Ws1r, bs1r, Ws2r, bs2r = refs[i:i + 4]; i += 4
        if not last:
            Wa1r, ba1r, Wa2r, ba2r = refs[i:i + 4]; i += 4
        norm_out, idx_out = refs[i:i + 2]

        k = pl.program_id(0)
        X = perm_ref[...]
        if rw != 0.0:
            X = X + rw * init_ref[...]
        nrm = jnp.sqrt(jnp.sum(X * X, axis=1, keepdims=True))
        h1 = _relu(_dot(X, Ws1r[0]) + bs1r[0, 0])
        t2 = _relu(_dot(h1, Ws2r[0]) + bs2r[0, 0])
        normalized = t2 / (nrm + 1e-6)
        norm_out[...] = normalized
        if last:
            flat = jnp.zeros((B, 1), jnp.int32)
        else:
            g1 = _relu(_dot(normalized, Wa1r[0]) + ba1r[0, 0])
            lg = _dot(g1, Wa2r[0]) + ba2r[0, 0]
            flat = _addr_flat(lg)
        valid_k = meta_ref[32 + k]
        rowv = jax.lax.broadcasted_iota(jnp.int32, (B, 1), 0)
        idx_out[...] = jnp.where(rowv < valid_k, flat, NB).reshape(1, B, 1)

    _ce = lambda m, k: jnp.minimum(jnp.maximum(m[k], 0), NB - 1)
    ex = lambda: pl.BlockSpec((1, EMB, EMB), lambda k, m: (_ce(m, k), 0, 0))
    exb = lambda: pl.BlockSpec((1, 1, EMB), lambda k, m: (_ce(m, k), 0, 0))
    exa = lambda: pl.BlockSpec((1, EMB, NCL), lambda k, m: (_ce(m, k), 0, 0))
    exab = lambda: pl.BlockSpec((1, 1, NCL), lambda k, m: (_ce(m, k), 0, 0))
    in_specs = [pl.BlockSpec((B, EMB), lambda k, m: (k, 0))]
    args = [perm]
    if rw != 0.0:
        in_specs.append(pl.BlockSpec((B, EMB), lambda k, m: (k, 0)))
        args.append(init_s)
    in_specs += [ex(), exb(), ex(), exb()]
    args += [Ws1, bs1.reshape(NB, 1, EMB), Ws2, bs2.reshape(NB, 1, EMB)]
    if not last:
        in_specs += [ex(), exb(), exa(), exab()]
        args += [Wa1, ba1.reshape(NB, 1, EMB), Wa2p, ba2p.reshape(NB, 1, NCL)]
    grid_spec = pltpu.PrefetchScalarGridSpec(
        num_scalar_prefetch=1,
        grid=(NT,),
        in_specs=in_specs,
        out_specs=[pl.BlockSpec((B, EMB), lambda k, m: (k, 0)),
                   pl.BlockSpec((1, B, 1), lambda k, m: (k, 0, 0))],
    )
    return pl.pallas_call(
        body,
        grid_spec=grid_spec,
        out_shape=[jax.ShapeDtypeStruct((SORT, EMB), jnp.float32),
                   jax.ShapeDtypeStruct((NT, B, 1), jnp.int32)],
    )(meta, *args)


def _head_body(fn_ref, init_ref, Wo1_ref, bo1_ref, Wo2_ref, bo2_ref, out_ref):
    final = fn_ref[...] + init_ref[...]
    h = _relu(_dot(final, Wo1_ref[...]) + bo1_ref[...])
    out_ref[...] = _dot(h, Wo2_ref[...]) + bo2_ref[...]


def _head(finalnorm, state, Wo1, bo1, Wo2, bo2):
    full = lambda r: pl.BlockSpec(None, lambda i: (0,) * r)
    return pl.pallas_call(
        _head_body,
        grid=(TOKENS // B,),
        in_specs=[pl.BlockSpec((B, EMB), lambda i: (i, 0)),
                  pl.BlockSpec((B, EMB), lambda i: (i, 0)),
                  full(2), full(1), full(2), full(1)],
        out_specs=pl.BlockSpec((B, NCL), lambda i: (i, 0)),
        out_shape=jax.ShapeDtypeStruct((TOKENS, NCL), jnp.float32),
    )(finalnorm, state, Wo1, bo1, Wo2, bo2)


_hist_t = functools.cache(lambda: _make_hist(TOKENS))
_hist_s = functools.cache(lambda: _make_hist(SORT))
_route0 = functools.cache(
    lambda: _make_route(TOKENS, gen_tokid=True, with_init=False))
_route_n = functools.cache(
    lambda: _make_route(SORT, gen_tokid=False, with_init=False))
_route_wi = functools.cache(
    lambda: _make_route(SORT, gen_tokid=False, with_init=True))
_unsort = functools.cache(_make_unsort)


def kernel(x, W_emb, b_emb, Ws1, bs1, Ws2, bs2, Wa1, ba1, Wa2, ba2, Wo1, bo1,
           Wo2, bo2):
    Wa2p = jnp.pad(Wa2, ((0, 0), (0, 0), (0, NCL - Wa2.shape[-1])))
    ba2p = jnp.pad(ba2, ((0, 0), (0, NCL - ba2.shape[-1])))

    state, idx3 = _embed(x, W_emb, b_emb, Wa1, ba1, Wa2p, ba2p)
    idx = idx3.reshape(TOKENS)

    perm, tok, meta = _route0()(idx, state, _hist_t()(idx))
    norm, i3 = _jump_call(meta, perm, None, Ws1, bs1, Ws2, bs2,
                          Wa1, ba1, Wa2p, ba2p, rw=0.0, last=False)

    idxs = i3.reshape(SORT)
    perm, tok, meta = _route_n()(idxs, norm, tok, _hist_s()(idxs))
    norm, i3 = _jump_call(meta, perm, None, Ws1, bs1, Ws2, bs2,
                          Wa1, ba1, Wa2p, ba2p, rw=0.0, last=False)

    for j in range(2, NJ + 1):
        idxs = i3.reshape(SORT)
        perm, tok, meta, init_s = _route_wi()(idxs, norm, tok, state,
                                              _hist_s()(idxs))
        norm, i3 = _jump_call(meta, perm, init_s, Ws1, bs1, Ws2, bs2,
                              Wa1, ba1, Wa2p, ba2p, rw=(j - 1) / (NJ - 1),
                              last=(j == NJ))

    finalnorm = _unsort()(i3.reshape(SORT), norm, tok)
    return _head(finalnorm[:TOKENS], state, Wo1, bo1, Wo2, bo2)


# speedup vs baseline: 1.0386x; 1.0386x over previous
import functools

import jax
import jax.numpy as jnp
from jax.experimental import pallas as pl
from jax.experimental.pallas import tpu as pltpu
from jax.experimental.pallas import tpu_sc as plsc

TOKENS = 2048
INPUT = 1024
EMB = 512
NB = 8
NJ = 4
NCL = 128
B = 128
NT = 24
SORT = NT * B
PADN = SORT + B
OUTPAD = TOKENS + B
TOKW = 128
NW = 16
L = 16

_relu = lambda z: jnp.maximum(z, 0.0)


def _dot(a, b):
    return jnp.dot(a, b, preferred_element_type=jnp.float32)


def _addr_flat(logits):
    a0 = (logits[:, 1:2] > logits[:, 0:1]).astype(jnp.int32)
    a1 = (logits[:, 3:4] > logits[:, 2:3]).astype(jnp.int32)
    a2 = (logits[:, 5:6] > logits[:, 4:5]).astype(jnp.int32)
    return 4 * a0 + 2 * a1 + a2


def _embed_body(x_ref, We_ref, be_ref, Wa1_ref, ba1_ref, Wa2_ref, ba2_ref,
                state_ref, idx_ref):
    state = _dot(x_ref[...], We_ref[...]) + be_ref[...]
    state_ref[...] = state
    h0 = _relu(_dot(state, Wa1_ref[0]) + ba1_ref[0])
    logits = _dot(h0, Wa2_ref[0]) + ba2_ref[0]
    idx_ref[...] = _addr_flat(logits).reshape(1, B, 1)


def _embed(x, W_emb, b_emb, Wa1, ba1, Wa2p, ba2p):
    full = lambda r: pl.BlockSpec(None, lambda i: (0,) * r)
    return pl.pallas_call(
        _embed_body,
        grid=(TOKENS // B,),
        in_specs=[pl.BlockSpec((B, INPUT), lambda i: (i, 0)),
                  full(2), full(1), full(3), full(2), full(3), full(2)],
        out_specs=[pl.BlockSpec((B, EMB), lambda i: (i, 0)),
                   pl.BlockSpec((1, B, 1), lambda i: (i, 0, 0))],
        out_shape=[jax.ShapeDtypeStruct((TOKENS, EMB), jnp.float32),
                   jax.ShapeDtypeStruct((TOKENS // B, B, 1), jnp.int32)],
    )(x, W_emb, b_emb, Wa1, ba1, Wa2p, ba2p)


def _lane_scalar(vec, e):
    iot = jax.lax.iota(jnp.int32, L)
    return jnp.sum(jnp.where(iot == e, vec, 0))


def _make_hist(n_in):
    nslots = n_in // NW
    nch = nslots // L
    scratch = [pltpu.VMEM((nslots,), jnp.int32),
               pltpu.VMEM((L,), jnp.int32)]

    def body(idx_hbm, ohist, idx_v, histbuf):
        wid = jax.lax.axis_index("s")
        base = wid * nslots
        iot = jax.lax.iota(jnp.int32, L)
        pltpu.sync_copy(idx_hbm.at[pl.ds(base, nslots)], idx_v)
        hist = jnp.zeros((L,), jnp.int32)
        for c in range(nch):
            v = idx_v[pl.ds(c * L, L)]
            for e in range(NB):
                cnt = jnp.sum((v == e).astype(jnp.int32))
                hist = hist + jnp.where(iot == e, cnt, 0)
        histbuf[...] = hist
        pltpu.sync_copy(histbuf, ohist.at[wid])

    mesh = plsc.VectorSubcoreMesh(core_axis_name="c", subcore_axis_name="s",
                                  num_cores=1)
    return pl.kernel(body,
                     out_type=jax.ShapeDtypeStruct((NW, L), jnp.int32),
                     mesh=mesh, scratch_types=tuple(scratch),
                     compiler_params=pltpu.CompilerParams(
                         needs_layout_passes=False))


def _make_route(n_in, gen_tokid, with_init):
    nslots = n_in // NW
    nch = nslots // L
    G = 1 if nslots <= 128 else 2
    gs = nslots // G
    gch = gs // L
    outs = [jax.ShapeDtypeStruct((PADN, EMB), jnp.float32),
            jax.ShapeDtypeStruct((PADN, TOKW), jnp.int32),
            jax.ShapeDtypeStruct((96,), jnp.int32)]
    if with_init:
        outs.append(jax.ShapeDtypeStruct((PADN, EMB), jnp.float32))
    scratch = [pltpu.VMEM((nslots,), jnp.int32),
               pltpu.VMEM((NW, L), jnp.int32),
               pltpu.VMEM((gs, EMB), jnp.float32),
               pltpu.VMEM((gs, TOKW), jnp.int32),
               pltpu.VMEM((96,), jnp.int32),
               [pltpu.VMEM((gs,), jnp.int32) for _ in range(G)],
               [pltpu.VMEM((gs,), jnp.int32) for _ in range(G)],
               [pltpu.SemaphoreType.DMA for _ in range(4)]]
    if with_init:
        scratch.append(pltpu.VMEM((gs, EMB), jnp.float32))

    def body(*refs):
        i = 0
        idx_hbm = refs[0]; rows_hbm = refs[1]; i = 2
        tok_hbm = None
        init_hbm = None
        if not gen_tokid:
            tok_hbm = refs[i]; i += 1
        if with_init:
            init_hbm = refs[i]; i += 1
        hists_hbm = refs[i]; i += 1
        operm = refs[i]; otok = refs[i + 1]; ometa = refs[i + 2]; i += 3
        if with_init:
            oinit = refs[i]; i += 1
        (idx_v, allhist, rowbuf, tokbuf, metabuf, posbufs, tidbufs,
         sems) = refs[i:i + 8]
        initbuf = refs[i + 8] if with_init else None

        wid = jax.lax.axis_index("s")
        base = wid * nslots
        iot = jax.lax.iota(jnp.int32, L)
        zer = jnp.zeros((L,), jnp.int32)

        pltpu.sync_copy(idx_hbm.at[pl.ds(base, nslots)], idx_v)
        pltpu.sync_copy(hists_hbm, allhist)

        g = jnp.zeros((L,), jnp.int32)
        pre = jnp.zeros((L,), jnp.int32)
        for w in range(NW):
            row = allhist[w, :]
            g = g + row
            pre = pre + jnp.where(jnp.int32(w) < wid, row, 0)

        lane8 = (iot < NB).astype(jnp.int32)
        counts = g * lane8
        T = ((counts + B - 1) >> 7) * lane8
        ts = plsc.cumsum(T) - T
        n_used = jnp.sum(T)
        mybase = ts * B + pre
        base_sc = [_lane_scalar(mybase, e) for e in range(NB)]
        dump = jnp.int32(SORT) + wid

        run = [jnp.int32(0)] * NB
        for gi in range(G):
            for c in range(gch):
                v = idx_v[pl.ds((gi * gch + c) * L, L)]
                pos = jnp.full((L,), 0, jnp.int32) + dump
                for e in range(NB):
                    m = v == e
                    mi = m.astype(jnp.int32)
                    r = plsc.cumsum(mi) - 1
                    pos = jnp.where(m, base_sc[e] + run[e] + r, pos)
                    run[e] = run[e] + jnp.sum(mi)
                posbufs[gi][pl.ds(c * L, L)] = jnp.clip(pos, 0, PADN - 1)

        @pl.when(wid == 0)
        def _():
            ts_sc = [_lane_scalar(ts, e) for e in range(NB)]
            cnt_sc = [_lane_scalar(counts, e) for e in range(NB)]
            k0 = iot
            k1 = iot + L
            e0 = jnp.full((L,), -1, jnp.int32)
            e1 = jnp.full((L,), -1, jnp.int32)
            for e in range(NB):
                e0 = e0 + (ts_sc[e] <= k0).astype(jnp.int32)
                e1 = e1 + (ts_sc[e] <= k1).astype(jnp.int32)
            e0 = jnp.clip(e0, 0, NB - 1)
            e1 = jnp.clip(e1, 0, NB - 1)
            v0 = zer
            v1 = zer
            for e in range(NB):
                c0 = jnp.clip(cnt_sc[e] - (k0 - ts_sc[e]) * B, 0, B)
                c1 = jnp.clip(cnt_sc[e] - (k1 - ts_sc[e]) * B, 0, B)
                v0 = jnp.where(e0 == e, c0, v0)
                v1 = jnp.where(e1 == e, c1, v1)
            metabuf[pl.ds(0, L)] = e0
            metabuf[pl.ds(16, L)] = e1
            metabuf[pl.ds(32, L)] = v0
            metabuf[pl.ds(48, L)] = v1
            metabuf[pl.ds(64, L)] = zer
            metabuf[pl.ds(80, L)] = zer + n_used
            pltpu.sync_copy(metabuf, ometa)

        for gi in range(G):
            gbase = base + gi * gs
            pltpu.sync_copy(rows_hbm.at[pl.ds(gbase, gs)], rowbuf)
            if gen_tokid:
                for c in range(gch):
                    tv = gbase + c * L + iot
                    plsc.store_scatter(tokbuf.at[pl.ds(c * L, L)],
                                       [iot, zer], tv)
            else:
                pltpu.sync_copy(tok_hbm.at[pl.ds(gbase, gs)], tokbuf)
            cp_rows = pltpu.async_copy(rowbuf, operm.at[posbufs[gi]], sems[0])
            cp_tok = pltpu.async_copy(tokbuf, otok.at[posbufs[gi]], sems[1])
            if with_init:
                for c in range(gch):
                    tv = plsc.load_gather(tokbuf.at[pl.ds(c * L, L)],
                                          [iot, zer])
                    vi = idx_v[pl.ds((gi * gch + c) * L, L)]
                    tidbufs[gi][pl.ds(c * L, L)] = jnp.clip(
                        jnp.where(vi < NB, tv, 0), 0, TOKENS - 1)
                pltpu.async_copy(init_hbm.at[tidbufs[gi]], initbuf,
                                 sems[2]).wait()
                cp_init = pltpu.async_copy(initbuf, oinit.at[posbufs[gi]],
                                           sems[3])
                cp_init.wait()
            cp_rows.wait()
            cp_tok.wait()

    mesh = plsc.VectorSubcoreMesh(core_axis_name="c", subcore_axis_name="s",
                                  num_cores=1)
    return pl.kernel(body, out_type=tuple(outs), mesh=mesh,
                     scratch_types=tuple(scratch),
                     compiler_params=pltpu.CompilerParams(
                         needs_layout_passes=False))


def _make_unsort():
    nslots = SORT // NW
    nch = nslots // L
    scratch = [pltpu.VMEM((nslots,), jnp.int32),
               pltpu.VMEM((L, EMB), jnp.float32),
               pltpu.VMEM((L, TOKW), jnp.int32),
               pltpu.VMEM((L,), jnp.int32),
               pltpu.SemaphoreType.DMA]

    def body(idx_hbm, rows_hbm, tok_hbm, out_hbm, idx_v, rowbuf, tokbuf,
             tidbuf, sem):
        wid = jax.lax.axis_index("s")
        base = wid * nslots
        iot = jax.lax.iota(jnp.int32, L)
        zer = jnp.zeros((L,), jnp.int32)
        pltpu.sync_copy(idx_hbm.at[pl.ds(base, nslots)], idx_v)
        for c in range(nch):
            v = idx_v[pl.ds(c * L, L)]
            pltpu.sync_copy(tok_hbm.at[pl.ds(base + c * L, L)], tokbuf)
            tv = plsc.load_gather(tokbuf, [iot, zer])
            tidbuf[...] = jnp.clip(jnp.where(v < NB, tv, jnp.int32(TOKENS) + wid),
                                   0, OUTPAD - 1)
            pltpu.sync_copy(rows_hbm.at[pl.ds(base + c * L, L)], rowbuf)
            pltpu.async_copy(rowbuf, out_hbm.at[tidbuf], sem).wait()

    mesh = plsc.VectorSubcoreMesh(core_axis_name="c", subcore_axis_name="s",
                                  num_cores=1)
    return pl.kernel(body,
                     out_type=jax.ShapeDtypeStruct((OUTPAD, EMB), jnp.float32),
                     mesh=mesh, scratch_types=tuple(scratch),
                     compiler_params=pltpu.CompilerParams(
                         needs_layout_passes=False))


def _jump_call(meta, perm, init_s, Ws1, bs1, Ws2, bs2, Wa1, ba1, Wa2p, ba2p,
               rw, last):
    def body(*refs):
        meta_ref = refs[0]
        i = 1
        perm_ref = refs[i]; i += 1
        init_ref = None
        if rw != 0.0:
            init_ref = refs[i]; i += 1
        Ws1r, bs1r, Ws2r, bs2r = refs[i:i + 4]; i += 4
        if not last:
            Wa1r, ba1r, Wa2r, ba2r = refs[i:i + 4]; i += 4
        norm_out, idx_out = refs[i:i + 2]

        k = pl.program_id(0)
        X = perm_ref[...]
        if rw != 0.0:
            X = X + rw * init_ref[...]
        nrm = jnp.sqrt(jnp.sum(X * X, axis=1, keepdims=True))
        h1 = _relu(_dot(X, Ws1r[0]) + bs1r[0, 0])
        t2 = _relu(_dot(h1, Ws2r[0]) + bs2r[0, 0])
        normalized = t2 / (nrm + 1e-6)
        norm_out[...] = normalized
        if last:
            flat = jnp.zeros((B, 1), jnp.int32)
        else:
            g1 = _relu(_dot(normalized, Wa1r[0]) + ba1r[0, 0])
            lg = _dot(g1, Wa2r[0]) + ba2r[0, 0]
            flat = _addr_flat(lg)
        valid_k = meta_ref[32 + k]
        rowv = jax.lax.broadcasted_iota(jnp.int32, (B, 1), 0)
        idx_out[...] = jnp.where(rowv < valid_k, flat, NB).reshape(1, B, 1)

    _ce = lambda m, k: jnp.minimum(jnp.maximum(m[k], 0), NB - 1)
    ex = lambda: pl.BlockSpec((1, EMB, EMB), lambda k, m: (_ce(m, k), 0, 0))
    exb = lambda: pl.BlockSpec((1, 1, EMB), lambda k, m: (_ce(m, k), 0, 0))
    exa = lambda: pl.BlockSpec((1, EMB, NCL), lambda k, m: (_ce(m, k), 0, 0))
    exab = lambda: pl.BlockSpec((1, 1, NCL), lambda k, m: (_ce(m, k), 0, 0))
    in_specs = [pl.BlockSpec((B, EMB), lambda k, m: (k, 0))]
    args = [perm]
    if rw != 0.0:
        in_specs.append(pl.BlockSpec((B, EMB), lambda k, m: (k, 0)))
        args.append(init_s)
    in_specs += [ex(), exb(), ex(), exb()]
    args += [Ws1, bs1.reshape(NB, 1, EMB), Ws2, bs2.reshape(NB, 1, EMB)]
    if not last:
        in_specs += [ex(), exb(), exa(), exab()]
        args += [Wa1, ba1.reshape(NB, 1, EMB), Wa2p, ba2p.reshape(NB, 1, NCL)]
    grid_spec = pltpu.PrefetchScalarGridSpec(
        num_scalar_prefetch=1,
        grid=(NT,),
        in_specs=in_specs,
        out_specs=[pl.BlockSpec((B, EMB), lambda k, m: (k, 0)),
                   pl.BlockSpec((1, B, 1), lambda k, m: (k, 0, 0))],
    )
    return pl.pallas_call(
        body,
        grid_spec=grid_spec,
        out_shape=[jax.ShapeDtypeStruct((SORT, EMB), jnp.float32),
                   jax.ShapeDtypeStruct((NT, B, 1), jnp.int32)],
    )(meta, *args)


def _head_body(fn_ref, init_ref, Wo1_ref, bo1_ref, Wo2_ref, bo2_ref, out_ref):
    final = fn_ref[...] + init_ref[...]
    h = _relu(_dot(final, Wo1_ref[...]) + bo1_ref[...])
    out_ref[...] = _dot(h, Wo2_ref[...]) + bo2_ref[...]


def _head(finalnorm, state, Wo1, bo1, Wo2, bo2):
    full = lambda r: pl.BlockSpec(None, lambda i: (0,) * r)
    return pl.pallas_call(
        _head_body,
        grid=(TOKENS // B,),
        in_specs=[pl.BlockSpec((B, EMB), lambda i: (i, 0)),
                  pl.BlockSpec((B, EMB), lambda i: (i, 0)),
                  full(2), full(1), full(2), full(1)],
        out_specs=pl.BlockSpec((B, NCL), lambda i: (i, 0)),
        out_shape=jax.ShapeDtypeStruct((TOKENS, NCL), jnp.float32),
    )(finalnorm, state, Wo1, bo1, Wo2, bo2)


_hist_t = functools.cache(lambda: _make_hist(TOKENS))
_hist_s = functools.cache(lambda: _make_hist(SORT))
_route0 = functools.cache(
    lambda: _make_route(TOKENS, gen_tokid=True, with_init=False))
_route_n = functools.cache(
    lambda: _make_route(SORT, gen_tokid=False, with_init=False))
_route_wi = functools.cache(
    lambda: _make_route(SORT, gen_tokid=False, with_init=True))
_unsort = functools.cache(_make_unsort)


def kernel(x, W_emb, b_emb, Ws1, bs1, Ws2, bs2, Wa1, ba1, Wa2, ba2, Wo1, bo1,
           Wo2, bo2):
    Wa2p = jnp.pad(Wa2, ((0, 0), (0, 0), (0, NCL - Wa2.shape[-1])))
    ba2p = jnp.pad(ba2, ((0, 0), (0, NCL - ba2.shape[-1])))

    state, idx3 = _embed(x, W_emb, b_emb, Wa1, ba1, Wa2p, ba2p)
    idx = idx3.reshape(TOKENS)

    perm, tok, meta = _route0()(idx, state, _hist_t()(idx))
    norm, i3 = _jump_call(meta, perm, None, Ws1, bs1, Ws2, bs2,
                          Wa1, ba1, Wa2p, ba2p, rw=0.0, last=False)

    idxs = i3.reshape(SORT)
    perm, tok, meta = _route_n()(idxs, norm, tok, _hist_s()(idxs))
    norm, i3 = _jump_call(meta, perm, None, Ws1, bs1, Ws2, bs2,
                          Wa1, ba1, Wa2p, ba2p, rw=0.0, last=False)

    for j in range(2, NJ + 1):
        idxs = i3.reshape(SORT)
        perm, tok, meta, init_s = _route_wi()(idxs, norm, tok, state,
                                              _hist_s()(idxs))
        norm, i3 = _jump_call(meta, perm, init_s, Ws1, bs1, Ws2, bs2,
                              Wa1, ba1, Wa2p, ba2p, rw=(j - 1) / (NJ - 1),
                              last=(j == NJ))

    finalnorm = _unsort()(i3.reshape(SORT), norm, tok)
    return _head(finalnorm[:TOKENS], state, Wo1, bo1, Wo2, bo2)
